# Initial kernel scaffold; baseline (speedup 1.0000x reference)
import jax
import jax.numpy as jnp
from jax.experimental import pallas as pl


def kernel(node_feat, edge_attr, edge_index, global_feat, edge_params, node_params, global_params):
    raise NotImplementedError("write your pallas kernel here")


# speedup vs baseline: 4.0855x; 4.0855x over previous
import functools

import jax
import jax.numpy as jnp
from jax import lax
from jax.experimental import pallas as pl
from jax.experimental.pallas import tpu as pltpu
from jax.experimental.pallas import tpu_sc as plsc

F32 = jnp.float32
NC = 2
NS = 16
LN_EPS = 1e-5


def _sc_mesh():
    return plsc.VectorSubcoreMesh(core_axis_name="c", subcore_axis_name="s")


def _make_counts(E, Nn):
    epw = E // (NC * NS)
    nv = Nn // 16

    def body(s_hbm, r_hbm, hs_out, hr_out, sidx, ridx, hs, hr):
        cid = lax.axis_index("c")
        sid = lax.axis_index("s")
        wid = sid * NC + cid
        base = wid * epw
        pltpu.sync_copy(s_hbm.at[pl.ds(base, epw)], sidx)
        pltpu.sync_copy(r_hbm.at[pl.ds(base, epw)], ridx)
        zeros = jnp.zeros((16,), F32)

        def zero_body(i, carry):
            hs[pl.ds(i * 16, 16)] = zeros
            hr[pl.ds(i * 16, 16)] = zeros
            return carry

        lax.fori_loop(0, nv, zero_body, 0)
        ones = jnp.ones((16,), F32)

        def acc_body(i, carry):
            si = sidx[pl.ds(i * 16, 16)]
            plsc.addupdate_scatter(hs, [si], ones)
            ri = ridx[pl.ds(i * 16, 16)]
            plsc.addupdate_scatter(hr, [ri], ones)
            return carry

        lax.fori_loop(0, epw // 16, acc_body, 0)
        pltpu.sync_copy(hs, hs_out.at[pl.ds(wid * Nn, Nn)])
        pltpu.sync_copy(hr, hr_out.at[pl.ds(wid * Nn, Nn)])

    return pl.kernel(
        body,
        out_type=[
            jax.ShapeDtypeStruct((NC * NS * Nn,), F32),
            jax.ShapeDtypeStruct((NC * NS * Nn,), F32),
        ],
        mesh=_sc_mesh(),
        scratch_types=[
            pltpu.VMEM((epw,), jnp.int32),
            pltpu.VMEM((epw,), jnp.int32),
            pltpu.VMEM((Nn,), F32),
            pltpu.VMEM((Nn,), F32),
        ],
        compiler_params=pltpu.CompilerParams(needs_layout_passes=False),
    )


def _make_gather(E, Nn, D):
    chunk = 256
    nch = E // chunk
    spt = 1000

    def body(ps_hbm, pr_hbm, s2_hbm, r2_hbm, gs_out, gr_out,
             tbl_sh, idx, rows, sem):
        cid = lax.axis_index("c")
        sid = lax.axis_index("s")

        @pl.when(sid < Nn // spt)
        def _():
            @pl.when(cid == 0)
            def _():
                pltpu.sync_copy(ps_hbm.at[pl.ds(sid * spt, spt)],
                                tbl_sh.at[pl.ds(sid * spt, spt)])

            @pl.when(cid != 0)
            def _():
                pltpu.sync_copy(pr_hbm.at[pl.ds(sid * spt, spt)],
                                tbl_sh.at[pl.ds(sid * spt, spt)])

        plsc.subcore_barrier()
        lo = (nch * sid) // NS
        hi = (nch * (sid + 1)) // NS

        def run(idx2_hbm, out_hbm):
            def chunk_body(i, carry):
                pltpu.sync_copy(idx2_hbm.at[pl.ds(i * (chunk // 128),
                                                  chunk // 128)], idx)
                descs = [
                    pltpu.async_copy(tbl_sh.at[idx.at[j]],
                                     rows.at[pl.ds(j * 128, 128)], sem)
                    for j in range(chunk // 128)
                ]
                for d in descs:
                    d.wait()
                pltpu.sync_copy(rows, out_hbm.at[pl.ds(i * chunk, chunk)])
                return carry

            lax.fori_loop(lo, hi, chunk_body, 0)

        @pl.when(cid == 0)
        def _():
            run(s2_hbm, gs_out)

        @pl.when(cid != 0)
        def _():
            run(r2_hbm, gr_out)

    return pl.kernel(
        body,
        out_type=[
            jax.ShapeDtypeStruct((E, D), F32),
            jax.ShapeDtypeStruct((E, D), F32),
        ],
        mesh=_sc_mesh(),
        scratch_types=[
            pltpu.VMEM_SHARED((Nn, D), F32),
            pltpu.VMEM((chunk // 128, 128), jnp.int32),
            pltpu.VMEM((chunk, D), F32),
            pltpu.SemaphoreType.DMA,
        ],
    )


def _make_scatter(E, Nn, D):
    chunk = 256
    nch = E // chunk
    spt = 1000

    def body(ne_hbm, r2_hbm, z_hbm, a0_out, a1_out, acc_sh, idx, rows):
        cid = lax.axis_index("c")
        sid = lax.axis_index("s")

        @pl.when(sid < Nn // spt)
        def _():
            pltpu.sync_copy(z_hbm.at[pl.ds(sid * spt, spt)],
                            acc_sh.at[pl.ds(sid * spt, spt)])

        plsc.subcore_barrier()
        ncc = nch // 2
        lo = (ncc * sid) // NS
        hi = (ncc * (sid + 1)) // NS

        def chunk_body(i, carry):
            gch = 2 * i + cid
            pltpu.sync_copy(r2_hbm.at[pl.ds(gch * (chunk // 128),
                                            chunk // 128)], idx)
            pltpu.sync_copy(ne_hbm.at[pl.ds(gch * chunk, chunk)], rows)
            for j in range(chunk // 128):
                pltpu.sync_copy(rows.at[pl.ds(j * 128, 128)],
                                acc_sh.at[idx.at[j]], add=True)
            return carry

        lax.fori_loop(lo, hi, chunk_body, 0)
        plsc.subcore_barrier()

        @pl.when(sid < Nn // spt)
        def _():
            @pl.when(cid == 0)
            def _():
                pltpu.sync_copy(acc_sh.at[pl.ds(sid * spt, spt)],
                                a0_out.at[pl.ds(sid * spt, spt)])

            @pl.when(cid != 0)
            def _():
                pltpu.sync_copy(acc_sh.at[pl.ds(sid * spt, spt)],
                                a1_out.at[pl.ds(sid * spt, spt)])

    return pl.kernel(
        body,
        out_type=[
            jax.ShapeDtypeStruct((Nn, D), F32),
            jax.ShapeDtypeStruct((Nn, D), F32),
        ],
        mesh=_sc_mesh(),
        scratch_types=[
            pltpu.VMEM_SHARED((Nn, D), F32),
            pltpu.VMEM((chunk // 128, 128), jnp.int32),
            pltpu.VMEM((chunk, D), F32),
        ],
    )


def _colstats_body(x_ref, su_ref, sq_ref):
    @pl.when(pl.program_id(0) == 0)
    def _():
        su_ref[...] = jnp.zeros_like(su_ref)
        sq_ref[...] = jnp.zeros_like(sq_ref)

    xv = x_ref[...]
    su_ref[...] += jnp.sum(xv, axis=0, keepdims=True)
    sq_ref[...] += jnp.sum(xv * xv, axis=0, keepdims=True)


def _colstats(x, tile):
    E, D = x.shape
    return pl.pallas_call(
        _colstats_body,
        grid=(E // tile,),
        in_specs=[pl.BlockSpec((tile, D), lambda i: (i, 0))],
        out_specs=[
            pl.BlockSpec((1, D), lambda i: (0, 0)),
            pl.BlockSpec((1, D), lambda i: (0, 0)),
        ],
        out_shape=[
            jax.ShapeDtypeStruct((1, D), F32),
            jax.ShapeDtypeStruct((1, D), F32),
        ],
        compiler_params=pltpu.CompilerParams(
            dimension_semantics=("arbitrary",)),
    )(x)


def _fold_body(E, nf_ref, hs_ref, hr_ref, easu_ref, easq_ref,
               bge_ref, bbe_ref, bgs_ref, bbs_ref, bgr_ref, bbr_ref,
               bbg_ref, w1e_ref, w1s_ref, w1r_ref, w1g_ref, b1_ref,
               ps_ref, pr_ref, ale_ref, cvec_ref):
    e_f = jnp.float32(E)
    nfv = nf_ref[...]
    cs = jnp.sum(hs_ref[...], axis=0, keepdims=True)
    cr = jnp.sum(hr_ref[...], axis=0, keepdims=True)

    def fold_block(cnt):
        m = jnp.dot(cnt, nfv, preferred_element_type=F32) / e_f
        q = jnp.dot(cnt, nfv * nfv, preferred_element_type=F32) / e_f
        return m, q - m * m

    m_s, v_s = fold_block(cs)
    m_r, v_r = fold_block(cr)
    a_s = bgs_ref[...] * lax.rsqrt(v_s + LN_EPS)
    d_s = bbs_ref[...] - m_s * a_s
    a_r = bgr_ref[...] * lax.rsqrt(v_r + LN_EPS)
    d_r = bbr_ref[...] - m_r * a_r
    m_e = easu_ref[...] / e_f
    v_e = easq_ref[...] / e_f - m_e * m_e
    a_e = bge_ref[...] * lax.rsqrt(v_e + LN_EPS)
    d_e = bbe_ref[...] - m_e * a_e
    ps_ref[...] = jnp.dot(nfv * a_s, w1s_ref[...], preferred_element_type=F32)
    pr_ref[...] = jnp.dot(nfv * a_r, w1r_ref[...], preferred_element_type=F32)
    ale_ref[...] = a_e
    cvec_ref[...] = (
        b1_ref[...]
        + jnp.dot(d_e, w1e_ref[...], preferred_element_type=F32)
        + jnp.dot(d_s, w1s_ref[...], preferred_element_type=F32)
        + jnp.dot(d_r, w1r_ref[...], preferred_element_type=F32)
        + jnp.dot(bbg_ref[...], w1g_ref[...], preferred_element_type=F32)
    )


def _layer_norm(h, lng, lnb):
    mu = jnp.mean(h, axis=-1, keepdims=True)
    var = jnp.mean((h - mu) ** 2, axis=-1, keepdims=True)
    return (h - mu) * lax.rsqrt(var + LN_EPS) * lng + lnb


def _edge_mlp_body(gs_ref, gr_ref, ea_ref, ale_ref, w1e_ref, cvec_ref,
                   w2_ref, b2_ref, lng_ref, lnb_ref, out_ref):
    z1 = gs_ref[...] + gr_ref[...] + cvec_ref[...]
    z1 = z1 + jnp.dot(ea_ref[...] * ale_ref[...], w1e_ref[...],
                      preferred_element_type=F32)
    h = jnp.maximum(z1, 0.0)
    h = jnp.dot(h, w2_ref[...], preferred_element_type=F32) + b2_ref[...]
    h = jnp.maximum(h, 0.0)
    out_ref[...] = _layer_norm(h, lng_ref[...], lnb_ref[...])


def _node_body(Nn, a0_ref, a1_ref, nf_ref,
               bga_ref, bba_ref, bgn_ref, bbn_ref, bbgg_ref,
               w1a_ref, w1n_ref, w1g_ref, b1_ref, w2_ref, b2_ref,
               lng_ref, lnb_ref,
               gbnb_ref, gw1_ref, gb1_ref, gw2_ref, gb2_ref,
               glng_ref, glnb_ref,
               nodes_ref, glob_ref):
    n_f = jnp.float32(Nn)
    agg = a0_ref[...] + a1_ref[...]
    nfv = nf_ref[...]

    def colstats(x):
        m = jnp.sum(x, axis=0, keepdims=True) / n_f
        q = jnp.sum(x * x, axis=0, keepdims=True) / n_f
        return m, q - m * m

    m_a, v_a = colstats(agg)
    m_n, v_n = colstats(nfv)
    a_a = bga_ref[...] * lax.rsqrt(v_a + LN_EPS)
    d_a = bba_ref[...] - m_a * a_a
    a_n = bgn_ref[...] * lax.rsqrt(v_n + LN_EPS)
    d_n = bbn_ref[...] - m_n * a_n
    cvec = (b1_ref[...]
            + jnp.dot(d_a, w1a_ref[...], preferred_element_type=F32)
            + jnp.dot(d_n, w1n_ref[...], preferred_element_type=F32)
            + jnp.dot(bbgg_ref[...], w1g_ref[...], preferred_element_type=F32))
    z = (jnp.dot(agg * a_a, w1a_ref[...], preferred_element_type=F32)
         + jnp.dot(nfv * a_n, w1n_ref[...], preferred_element_type=F32)
         + cvec)
    h = jnp.maximum(z, 0.0)
    h = jnp.dot(h, w2_ref[...], preferred_element_type=F32) + b2_ref[...]
    h = jnp.maximum(h, 0.0)
    nodes_ref[...] = _layer_norm(h, lng_ref[...], lnb_ref[...])

    hg = jnp.maximum(
        jnp.dot(gbnb_ref[...], gw1_ref[...], preferred_element_type=F32)
        + gb1_ref[...], 0.0)
    hg = jnp.maximum(
        jnp.dot(hg, gw2_ref[...], preferred_element_type=F32)
        + gb2_ref[...], 0.0)
    glob_ref[...] = _layer_norm(hg, glng_ref[...], glnb_ref[...])


def _full_spec(shape):
    return pl.BlockSpec(shape, lambda *_: tuple(0 for _ in shape))


def kernel(node_feat, edge_attr, edge_index, global_feat,
           edge_params, node_params, global_params):
    Nn, DN = node_feat.shape
    E, DE = edge_attr.shape
    DG = global_feat.shape[-1]
    D = DN

    s = edge_index[0]
    r = edge_index[1]
    s2 = s.reshape(E // 128, 128)
    r2 = r.reshape(E // 128, 128)

    ep, np_, gp = edge_params, node_params, global_params
    row = lambda a: a.reshape(1, -1)

    hist_s, hist_r = _make_counts(E, Nn)(s, r)
    hist_s = hist_s.reshape(NC * NS, Nn)
    hist_r = hist_r.reshape(NC * NS, Nn)
    easu, easq = _colstats(edge_attr, 4000)

    w1 = ep["W1"]
    fold = pl.pallas_call(
        functools.partial(_fold_body, E),
        in_specs=[
            _full_spec((Nn, DN)), _full_spec((NC * NS, Nn)),
            _full_spec((NC * NS, Nn)),
            _full_spec((1, DE)), _full_spec((1, DE)),
            _full_spec((1, DE)), _full_spec((1, DE)),
            _full_spec((1, DN)), _full_spec((1, DN)),
            _full_spec((1, DN)), _full_spec((1, DN)),
            _full_spec((1, DG)),
            _full_spec((DE, D)), _full_spec((DN, D)),
            _full_spec((DN, D)), _full_spec((DG, D)),
            _full_spec((1, D)),
        ],
        out_specs=[
            _full_spec((Nn, D)), _full_spec((Nn, D)),
            _full_spec((1, DE)), _full_spec((1, D)),
        ],
        out_shape=[
            jax.ShapeDtypeStruct((Nn, D), F32),
            jax.ShapeDtypeStruct((Nn, D), F32),
            jax.ShapeDtypeStruct((1, DE), F32),
            jax.ShapeDtypeStruct((1, D), F32),
        ],
    )
    p_s, p_r, al_e, cvec = fold(
        node_feat, hist_s, hist_r, easu, easq,
        row(ep["bn_g"][:DE]), row(ep["bn_b"][:DE]),
        row(ep["bn_g"][DE:DE + DN]), row(ep["bn_b"][DE:DE + DN]),
        row(ep["bn_g"][DE + DN:DE + 2 * DN]),
        row(ep["bn_b"][DE + DN:DE + 2 * DN]),
        row(ep["bn_b"][DE + 2 * DN:]),
        w1[:DE], w1[DE:DE + DN], w1[DE + DN:DE + 2 * DN], w1[DE + 2 * DN:],
        row(ep["b1"]),
    )

    gs_arr, gr_arr = _make_gather(E, Nn, D)(p_s, p_r, s2, r2)

    TE = 2000
    edge_call = pl.pallas_call(
        _edge_mlp_body,
        grid=(E // TE,),
        in_specs=[
            pl.BlockSpec((TE, D), lambda i: (i, 0)),
            pl.BlockSpec((TE, D), lambda i: (i, 0)),
            pl.BlockSpec((TE, DE), lambda i: (i, 0)),
            pl.BlockSpec((1, DE), lambda i: (0, 0)),
            pl.BlockSpec((DE, D), lambda i: (0, 0)),
            pl.BlockSpec((1, D), lambda i: (0, 0)),
            pl.BlockSpec((D, D), lambda i: (0, 0)),
            pl.BlockSpec((1, D), lambda i: (0, 0)),
            pl.BlockSpec((1, D), lambda i: (0, 0)),
            pl.BlockSpec((1, D), lambda i: (0, 0)),
        ],
        out_specs=pl.BlockSpec((TE, D), lambda i: (i, 0)),
        out_shape=jax.ShapeDtypeStruct((E, D), F32),
        compiler_params=pltpu.CompilerParams(
            dimension_semantics=("arbitrary",)),
    )
    new_edges = edge_call(
        gs_arr, gr_arr, edge_attr, al_e, w1[:DE], cvec,
        ep["W2"], row(ep["b2"]), row(ep["ln_g"]), row(ep["ln_b"]))

    zsrc = jnp.zeros((Nn, D), F32)
    a0, a1 = _make_scatter(E, Nn, D)(new_edges, r2, zsrc)

    w1n = np_["W1"]
    NODE_IN = D + DN + DG
    GLOBAL_IN = gp["W1"].shape[0]
    node_call = pl.pallas_call(
        functools.partial(_node_body, Nn),
        in_specs=[
            _full_spec((Nn, D)), _full_spec((Nn, D)), _full_spec((Nn, DN)),
            _full_spec((1, D)), _full_spec((1, D)),
            _full_spec((1, DN)), _full_spec((1, DN)),
            _full_spec((1, DG)),
            _full_spec((D, D)), _full_spec((DN, D)), _full_spec((DG, D)),
            _full_spec((1, D)), _full_spec((D, D)), _full_spec((1, D)),
            _full_spec((1, D)), _full_spec((1, D)),
            _full_spec((1, GLOBAL_IN)), _full_spec((GLOBAL_IN, D)),
            _full_spec((1, D)), _full_spec((D, D)), _full_spec((1, D)),
            _full_spec((1, D)), _full_spec((1, D)),
        ],
        out_specs=[
            _full_spec((Nn, D)),
            _full_spec((1, D)),
        ],
        out_shape=[
            jax.ShapeDtypeStruct((Nn, D), F32),
            jax.ShapeDtypeStruct((1, D), F32),
        ],
    )
    new_nodes, new_global = node_call(
        a0, a1, node_feat,
        row(np_["bn_g"][:D]), row(np_["bn_b"][:D]),
        row(np_["bn_g"][D:D + DN]), row(np_["bn_b"][D:D + DN]),
        row(np_["bn_b"][D + DN:]),
        w1n[:D], w1n[D:D + DN], w1n[D + DN:],
        row(np_["b1"]), np_["W2"], row(np_["b2"]),
        row(np_["ln_g"]), row(np_["ln_b"]),
        row(gp["bn_b"]), gp["W1"], row(gp["b1"]), gp["W2"], row(gp["b2"]),
        row(gp["ln_g"]), row(gp["ln_b"]),
    )
    return (new_nodes, new_edges, new_global)


# speedup vs baseline: 5.1813x; 1.2682x over previous
import functools

import jax
import jax.numpy as jnp
from jax import lax
from jax.experimental import pallas as pl
from jax.experimental.pallas import tpu as pltpu
from jax.experimental.pallas import tpu_sc as plsc

F32 = jnp.float32
NC = 2
NS = 16
LN_EPS = 1e-5


def _sc_mesh():
    return plsc.VectorSubcoreMesh(core_axis_name="c", subcore_axis_name="s")


def _make_counts(E, Nn):
    epw = E // (NC * NS)
    nv = Nn // 16

    def body(s_hbm, r_hbm, hs_out, hr_out, sidx, ridx, hs, hr):
        cid = lax.axis_index("c")
        sid = lax.axis_index("s")
        wid = sid * NC + cid
        base = wid * epw
        pltpu.sync_copy(s_hbm.at[pl.ds(base, epw)], sidx)
        pltpu.sync_copy(r_hbm.at[pl.ds(base, epw)], ridx)
        zeros = jnp.zeros((16,), F32)

        def zero_body(i, carry):
            hs[pl.ds(i * 16, 16)] = zeros
            hr[pl.ds(i * 16, 16)] = zeros
            return carry

        lax.fori_loop(0, nv, zero_body, 0)
        ones = jnp.ones((16,), F32)

        def acc_body(i, carry):
            si = sidx[pl.ds(i * 16, 16)]
            plsc.addupdate_scatter(hs, [si], ones)
            ri = ridx[pl.ds(i * 16, 16)]
            plsc.addupdate_scatter(hr, [ri], ones)
            return carry

        lax.fori_loop(0, epw // 16, acc_body, 0)
        pltpu.sync_copy(hs, hs_out.at[pl.ds(wid * Nn, Nn)])
        pltpu.sync_copy(hr, hr_out.at[pl.ds(wid * Nn, Nn)])

    return pl.kernel(
        body,
        out_type=[
            jax.ShapeDtypeStruct((NC * NS * Nn,), F32),
            jax.ShapeDtypeStruct((NC * NS * Nn,), F32),
        ],
        mesh=_sc_mesh(),
        scratch_types=[
            pltpu.VMEM((epw,), jnp.int32),
            pltpu.VMEM((epw,), jnp.int32),
            pltpu.VMEM((Nn,), F32),
            pltpu.VMEM((Nn,), F32),
        ],
        compiler_params=pltpu.CompilerParams(needs_layout_passes=False),
    )


def _make_gather(E, Nn, D):
    nch = E // 128
    spt = 1000

    def body(ps_hbm, pr_hbm, s_hbm, r_hbm, gs_out, gr_out,
             tbl_sh, idx0, idx1, rows0, rows1, sem0, sem1):
        cid = lax.axis_index("c")
        sid = lax.axis_index("s")

        @pl.when(sid < Nn // spt)
        def _():
            @pl.when(cid == 0)
            def _():
                pltpu.sync_copy(ps_hbm.at[pl.ds(sid * spt, spt)],
                                tbl_sh.at[pl.ds(sid * spt, spt)])

            @pl.when(cid != 0)
            def _():
                pltpu.sync_copy(pr_hbm.at[pl.ds(sid * spt, spt)],
                                tbl_sh.at[pl.ds(sid * spt, spt)])

        plsc.subcore_barrier()
        lo = (nch * sid) // NS
        hi = (nch * (sid + 1)) // NS
        n = hi - lo
        rows = (rows0, rows1)
        idxs = (idx0, idx1)
        sems = (sem0, sem1)

        def run(idx_hbm, out_hbm):
            pltpu.sync_copy(idx_hbm.at[pl.ds(lo * 128, 128)], idx0)

            @pl.when(n > 0)
            def _():
                pltpu.async_copy(tbl_sh.at[idx0], rows0, sem0)

            def pair(i, carry):
                for b in range(2):
                    c = 2 * i + b

                    @pl.when(c < n)
                    def _():
                        @pl.when(c + 1 < n)
                        def _():
                            pltpu.sync_copy(
                                idx_hbm.at[pl.ds((lo + c + 1) * 128, 128)],
                                idxs[1 - b])
                            pltpu.async_copy(tbl_sh.at[idxs[1 - b]],
                                             rows[1 - b], sems[1 - b])

                        pltpu.make_async_copy(
                            tbl_sh.at[idxs[b]], rows[b], sems[b]).wait()
                        pltpu.sync_copy(
                            rows[b], out_hbm.at[pl.ds((lo + c) * 128, 128)])
                return carry

            lax.fori_loop(0, (n + 1) // 2, pair, 0)

        @pl.when(cid == 0)
        def _():
            run(s_hbm, gs_out)

        @pl.when(cid != 0)
        def _():
            run(r_hbm, gr_out)

    return pl.kernel(
        body,
        out_type=[
            jax.ShapeDtypeStruct((E, D), F32),
            jax.ShapeDtypeStruct((E, D), F32),
        ],
        mesh=_sc_mesh(),
        scratch_types=[
            pltpu.VMEM_SHARED((Nn, D), F32),
            pltpu.VMEM((128,), jnp.int32),
            pltpu.VMEM((128,), jnp.int32),
            pltpu.VMEM((128, D), F32),
            pltpu.VMEM((128, D), F32),
            pltpu.SemaphoreType.DMA,
            pltpu.SemaphoreType.DMA,
        ],
    )


def _make_scatter(E, Nn, D):
    nch = E // 128
    spt = 1000
    CAP = 79

    def body(ne_hbm, r_hbm, z_hbm, a0_out, a1_out,
             acc_sh, idx0, idx1, rows0, rows1, sem0, sem1):
        cid = lax.axis_index("c")
        sid = lax.axis_index("s")

        @pl.when(sid < Nn // spt)
        def _():
            pltpu.sync_copy(z_hbm.at[pl.ds(sid * spt, spt)],
                            acc_sh.at[pl.ds(sid * spt, spt)])

        plsc.subcore_barrier()
        ncc = nch // 2
        lo = cid * ncc + (ncc * sid) // NS
        hi = cid * ncc + (ncc * (sid + 1)) // NS
        n = hi - lo
        rows = (rows0, rows1)
        idxs = (idx0, idx1)
        sems = (sem0, sem1)

        pltpu.sync_copy(r_hbm.at[pl.ds(lo * 128, 128)], idx0)

        @pl.when(n > 0)
        def _():
            pltpu.async_copy(ne_hbm.at[pl.ds(lo * 128, 128)], rows0, sem0)

        def pair(i, carry):
            for b in range(2):
                c = 2 * i + b

                @pl.when(c < n)
                def _():
                    @pl.when(c + 1 < n)
                    def _():
                        pltpu.sync_copy(
                            r_hbm.at[pl.ds((lo + c + 1) * 128, 128)],
                            idxs[1 - b])
                        pltpu.async_copy(
                            ne_hbm.at[pl.ds((lo + c + 1) * 128, 128)],
                            rows[1 - b], sems[1 - b])

                    pltpu.make_async_copy(
                        ne_hbm.at[pl.ds((lo + c) * 128, 128)], rows[b],
                        sems[b]).wait()
                    pltpu.sync_copy(rows[b], acc_sh.at[idxs[b]], add=True)
            return carry

        lax.fori_loop(0, (n + 1) // 2, pair, 0)
        plsc.subcore_barrier()

        @pl.when(sid < Nn // spt)
        def _():
            @pl.when(cid == 0)
            def _():
                pltpu.sync_copy(acc_sh.at[pl.ds(sid * spt, spt)],
                                a0_out.at[pl.ds(sid * spt, spt)])

            @pl.when(cid != 0)
            def _():
                pltpu.sync_copy(acc_sh.at[pl.ds(sid * spt, spt)],
                                a1_out.at[pl.ds(sid * spt, spt)])

    return pl.kernel(
        body,
        out_type=[
            jax.ShapeDtypeStruct((Nn, D), F32),
            jax.ShapeDtypeStruct((Nn, D), F32),
        ],
        mesh=_sc_mesh(),
        scratch_types=[
            pltpu.VMEM_SHARED((Nn, D), F32),
            pltpu.VMEM((128,), jnp.int32),
            pltpu.VMEM((128,), jnp.int32),
            pltpu.VMEM((128, D), F32),
            pltpu.VMEM((128, D), F32),
            pltpu.SemaphoreType.DMA,
            pltpu.SemaphoreType.DMA,
        ],
    )


def _colstats_body(x_ref, su_ref, sq_ref):
    @pl.when(pl.program_id(0) == 0)
    def _():
        su_ref[...] = jnp.zeros_like(su_ref)
        sq_ref[...] = jnp.zeros_like(sq_ref)

    xv = x_ref[...]
    su_ref[...] += jnp.sum(xv, axis=0, keepdims=True)
    sq_ref[...] += jnp.sum(xv * xv, axis=0, keepdims=True)


def _colstats(x, tile):
    E, D = x.shape
    return pl.pallas_call(
        _colstats_body,
        grid=(E // tile,),
        in_specs=[pl.BlockSpec((tile, D), lambda i: (i, 0))],
        out_specs=[
            pl.BlockSpec((1, D), lambda i: (0, 0)),
            pl.BlockSpec((1, D), lambda i: (0, 0)),
        ],
        out_shape=[
            jax.ShapeDtypeStruct((1, D), F32),
            jax.ShapeDtypeStruct((1, D), F32),
        ],
        compiler_params=pltpu.CompilerParams(
            dimension_semantics=("arbitrary",)),
    )(x)


def _fold_body(E, nf_ref, hs_ref, hr_ref, easu_ref, easq_ref,
               bge_ref, bbe_ref, bgs_ref, bbs_ref, bgr_ref, bbr_ref,
               bbg_ref, w1e_ref, w1s_ref, w1r_ref, w1g_ref, b1_ref,
               ps_ref, pr_ref, ale_ref, cvec_ref):
    e_f = jnp.float32(E)
    nfv = nf_ref[...]
    cs = jnp.sum(hs_ref[...], axis=0, keepdims=True)
    cr = jnp.sum(hr_ref[...], axis=0, keepdims=True)

    def fold_block(cnt):
        m = jnp.dot(cnt, nfv, preferred_element_type=F32) / e_f
        q = jnp.dot(cnt, nfv * nfv, preferred_element_type=F32) / e_f
        return m, q - m * m

    m_s, v_s = fold_block(cs)
    m_r, v_r = fold_block(cr)
    a_s = bgs_ref[...] * lax.rsqrt(v_s + LN_EPS)
    d_s = bbs_ref[...] - m_s * a_s
    a_r = bgr_ref[...] * lax.rsqrt(v_r + LN_EPS)
    d_r = bbr_ref[...] - m_r * a_r
    m_e = easu_ref[...] / e_f
    v_e = easq_ref[...] / e_f - m_e * m_e
    a_e = bge_ref[...] * lax.rsqrt(v_e + LN_EPS)
    d_e = bbe_ref[...] - m_e * a_e
    ps_ref[...] = jnp.dot(nfv * a_s, w1s_ref[...], preferred_element_type=F32)
    pr_ref[...] = jnp.dot(nfv * a_r, w1r_ref[...], preferred_element_type=F32)
    ale_ref[...] = a_e
    cvec_ref[...] = (
        b1_ref[...]
        + jnp.dot(d_e, w1e_ref[...], preferred_element_type=F32)
        + jnp.dot(d_s, w1s_ref[...], preferred_element_type=F32)
        + jnp.dot(d_r, w1r_ref[...], preferred_element_type=F32)
        + jnp.dot(bbg_ref[...], w1g_ref[...], preferred_element_type=F32)
    )


def _layer_norm(h, lng, lnb):
    mu = jnp.mean(h, axis=-1, keepdims=True)
    var = jnp.mean((h - mu) ** 2, axis=-1, keepdims=True)
    return (h - mu) * lax.rsqrt(var + LN_EPS) * lng + lnb


def _edge_mlp_body(gs_ref, gr_ref, ea8_ref, w1big_ref, cvec_ref,
                   w2_ref, b2_ref, lng_ref, lnb_ref, out_ref):
    te = gs_ref.shape[0]
    z_ea = jnp.dot(ea8_ref[...], w1big_ref[...],
                   preferred_element_type=F32).reshape(te, -1)
    z1 = gs_ref[...] + gr_ref[...] + cvec_ref[...] + z_ea
    h = jnp.maximum(z1, 0.0)
    h = jnp.dot(h, w2_ref[...], preferred_element_type=F32) + b2_ref[...]
    h = jnp.maximum(h, 0.0)
    out_ref[...] = _layer_norm(h, lng_ref[...], lnb_ref[...])


def _node_body(Nn, a0_ref, a1_ref, nf_ref,
               bga_ref, bba_ref, bgn_ref, bbn_ref, bbgg_ref,
               w1a_ref, w1n_ref, w1g_ref, b1_ref, w2_ref, b2_ref,
               lng_ref, lnb_ref,
               gbnb_ref, gw1_ref, gb1_ref, gw2_ref, gb2_ref,
               glng_ref, glnb_ref,
               nodes_ref, glob_ref):
    n_f = jnp.float32(Nn)
    agg = a0_ref[...] + a1_ref[...]
    nfv = nf_ref[...]

    def colstats(x):
        m = jnp.sum(x, axis=0, keepdims=True) / n_f
        q = jnp.sum(x * x, axis=0, keepdims=True) / n_f
        return m, q - m * m

    m_a, v_a = colstats(agg)
    m_n, v_n = colstats(nfv)
    a_a = bga_ref[...] * lax.rsqrt(v_a + LN_EPS)
    d_a = bba_ref[...] - m_a * a_a
    a_n = bgn_ref[...] * lax.rsqrt(v_n + LN_EPS)
    d_n = bbn_ref[...] - m_n * a_n
    cvec = (b1_ref[...]
            + jnp.dot(d_a, w1a_ref[...], preferred_element_type=F32)
            + jnp.dot(d_n, w1n_ref[...], preferred_element_type=F32)
            + jnp.dot(bbgg_ref[...], w1g_ref[...], preferred_element_type=F32))
    z = (jnp.dot(agg * a_a, w1a_ref[...], preferred_element_type=F32)
         + jnp.dot(nfv * a_n, w1n_ref[...], preferred_element_type=F32)
         + cvec)
    h = jnp.maximum(z, 0.0)
    h = jnp.dot(h, w2_ref[...], preferred_element_type=F32) + b2_ref[...]
    h = jnp.maximum(h, 0.0)
    nodes_ref[...] = _layer_norm(h, lng_ref[...], lnb_ref[...])

    hg = jnp.maximum(
        jnp.dot(gbnb_ref[...], gw1_ref[...], preferred_element_type=F32)
        + gb1_ref[...], 0.0)
    hg = jnp.maximum(
        jnp.dot(hg, gw2_ref[...], preferred_element_type=F32)
        + gb2_ref[...], 0.0)
    glob_ref[...] = _layer_norm(hg, glng_ref[...], glnb_ref[...])


def _full_spec(shape):
    return pl.BlockSpec(shape, lambda *_: tuple(0 for _ in shape))


def kernel(node_feat, edge_attr, edge_index, global_feat,
           edge_params, node_params, global_params):
    Nn, DN = node_feat.shape
    E, DE = edge_attr.shape
    DG = global_feat.shape[-1]
    D = DN

    s = edge_index[0]
    r = edge_index[1]

    ep, np_, gp = edge_params, node_params, global_params
    row = lambda a: a.reshape(1, -1)

    hist_s, hist_r = _make_counts(E, Nn)(s, r)
    hist_s = hist_s.reshape(NC * NS, Nn)
    hist_r = hist_r.reshape(NC * NS, Nn)
    grp = 128 // DE
    ea8 = edge_attr.reshape(E // grp, 128)
    easu128, easq128 = _colstats(ea8, 4000)
    easu = easu128.reshape(grp, DE).sum(axis=0).reshape(1, DE)
    easq = easq128.reshape(grp, DE).sum(axis=0).reshape(1, DE)

    w1 = ep["W1"]
    fold = pl.pallas_call(
        functools.partial(_fold_body, E),
        in_specs=[
            _full_spec((Nn, DN)), _full_spec((NC * NS, Nn)),
            _full_spec((NC * NS, Nn)),
            _full_spec((1, DE)), _full_spec((1, DE)),
            _full_spec((1, DE)), _full_spec((1, DE)),
            _full_spec((1, DN)), _full_spec((1, DN)),
            _full_spec((1, DN)), _full_spec((1, DN)),
            _full_spec((1, DG)),
            _full_spec((DE, D)), _full_spec((DN, D)),
            _full_spec((DN, D)), _full_spec((DG, D)),
            _full_spec((1, D)),
        ],
        out_specs=[
            _full_spec((Nn, D)), _full_spec((Nn, D)),
            _full_spec((1, DE)), _full_spec((1, D)),
        ],
        out_shape=[
            jax.ShapeDtypeStruct((Nn, D), F32),
            jax.ShapeDtypeStruct((Nn, D), F32),
            jax.ShapeDtypeStruct((1, DE), F32),
            jax.ShapeDtypeStruct((1, D), F32),
        ],
    )
    p_s, p_r, al_e, cvec = fold(
        node_feat, hist_s, hist_r, easu, easq,
        row(ep["bn_g"][:DE]), row(ep["bn_b"][:DE]),
        row(ep["bn_g"][DE:DE + DN]), row(ep["bn_b"][DE:DE + DN]),
        row(ep["bn_g"][DE + DN:DE + 2 * DN]),
        row(ep["bn_b"][DE + DN:DE + 2 * DN]),
        row(ep["bn_b"][DE + 2 * DN:]),
        w1[:DE], w1[DE:DE + DN], w1[DE + DN:DE + 2 * DN], w1[DE + 2 * DN:],
        row(ep["b1"]),
    )

    gs_arr, gr_arr = _make_gather(E, Nn, D)(p_s, p_r, s, r)

    scaled_w1e = al_e.reshape(DE, 1) * w1[:DE]
    w1big = jnp.einsum("ab,io->aibo", jnp.eye(grp, dtype=F32),
                       scaled_w1e).reshape(128, grp * D)
    TE = 3200
    edge_call = pl.pallas_call(
        _edge_mlp_body,
        grid=(E // TE,),
        in_specs=[
            pl.BlockSpec((TE, D), lambda i: (i, 0)),
            pl.BlockSpec((TE, D), lambda i: (i, 0)),
            pl.BlockSpec((TE // grp, 128), lambda i: (i, 0)),
            pl.BlockSpec((128, grp * D), lambda i: (0, 0)),
            pl.BlockSpec((1, D), lambda i: (0, 0)),
            pl.BlockSpec((D, D), lambda i: (0, 0)),
            pl.BlockSpec((1, D), lambda i: (0, 0)),
            pl.BlockSpec((1, D), lambda i: (0, 0)),
            pl.BlockSpec((1, D), lambda i: (0, 0)),
        ],
        out_specs=pl.BlockSpec((TE, D), lambda i: (i, 0)),
        out_shape=jax.ShapeDtypeStruct((E, D), F32),
        compiler_params=pltpu.CompilerParams(
            dimension_semantics=("arbitrary",)),
    )
    new_edges = edge_call(
        gs_arr, gr_arr, ea8, w1big, cvec,
        ep["W2"], row(ep["b2"]), row(ep["ln_g"]), row(ep["ln_b"]))

    zsrc = jnp.zeros((Nn, D), F32)
    a0, a1 = _make_scatter(E, Nn, D)(new_edges, r, zsrc)

    w1n = np_["W1"]
    NODE_IN = D + DN + DG
    GLOBAL_IN = gp["W1"].shape[0]
    node_call = pl.pallas_call(
        functools.partial(_node_body, Nn),
        in_specs=[
            _full_spec((Nn, D)), _full_spec((Nn, D)), _full_spec((Nn, DN)),
            _full_spec((1, D)), _full_spec((1, D)),
            _full_spec((1, DN)), _full_spec((1, DN)),
            _full_spec((1, DG)),
            _full_spec((D, D)), _full_spec((DN, D)), _full_spec((DG, D)),
            _full_spec((1, D)), _full_spec((D, D)), _full_spec((1, D)),
            _full_spec((1, D)), _full_spec((1, D)),
            _full_spec((1, GLOBAL_IN)), _full_spec((GLOBAL_IN, D)),
            _full_spec((1, D)), _full_spec((D, D)), _full_spec((1, D)),
            _full_spec((1, D)), _full_spec((1, D)),
        ],
        out_specs=[
            _full_spec((Nn, D)),
            _full_spec((1, D)),
        ],
        out_shape=[
            jax.ShapeDtypeStruct((Nn, D), F32),
            jax.ShapeDtypeStruct((1, D), F32),
        ],
    )
    new_nodes, new_global = node_call(
        a0, a1, node_feat,
        row(np_["bn_g"][:D]), row(np_["bn_b"][:D]),
        row(np_["bn_g"][D:D + DN]), row(np_["bn_b"][D:D + DN]),
        row(np_["bn_b"][D + DN:]),
        w1n[:D], w1n[D:D + DN], w1n[D + DN:],
        row(np_["b1"]), np_["W2"], row(np_["b2"]),
        row(np_["ln_g"]), row(np_["ln_b"]),
        row(gp["bn_b"]), gp["W1"], row(gp["b1"]), gp["W2"], row(gp["b2"]),
        row(gp["ln_g"]), row(gp["ln_b"]),
    )
    return (new_nodes, new_edges, new_global)
